# baseline (speedup 1.0000x reference)
import functools

import jax
import jax.numpy as jnp
from jax import lax
from jax.experimental import pallas as pl
from jax.experimental.pallas import tpu as pltpu
from jax.experimental.pallas import tpu_sc as plsc

N = 2226
E = 6525
R = 1800

NC = 2
NS = 16
NW = NC * NS

PER_R = 64
PER_E = 208
PER_N = 80

NP = 2304
EP = 6528

S_NODE = PER_N + 8
S_EDGE = PER_E + 8
S_LENE = PER_E + 5
S_ROPE = PER_N + 2
S_REFL = PER_R

O_POS = 0
O_ACT = O_POS + 3 * NP
O_DIR = O_ACT + 3 * NP
O_STR = O_DIR + 3 * NP
O_EDGE = O_STR + NP
O_CONST = O_EDGE + 2 * EP
S_CONST = 400
F_TOTAL = O_CONST + 512

_F32 = jnp.float32
_I32 = jnp.int32


def _sqrt16(ss):
    i = lax.bitcast_convert_type(ss, _I32)
    y = lax.bitcast_convert_type(
        jnp.int32(0x5F3759DF) - lax.shift_right_logical(i, 1), _F32)
    for _ in range(3):
        y = y * (1.5 - 0.5 * ss * y * y)
    return jnp.where(ss > 0.0, ss * y, 0.0)


def _pack_body(pos_ref, str_ref, bias_ref, rotm_ref, foc_ref, act_ref,
               dir_ref, edge_ref, f_ref):
    post = pos_ref[...].T
    actt = act_ref[...].T
    dirt = dir_ref[...].T
    strt = str_ref[...].T
    edgt = edge_ref[...].astype(_F32).T
    for c in range(3):
        f_ref[pl.ds(O_POS + c * NP, N)] = post[c, :]
        f_ref[pl.ds(O_ACT + c * NP, N)] = actt[c, :]
        f_ref[pl.ds(O_DIR + c * NP, N)] = dirt[c, :]
    f_ref[pl.ds(O_STR, N)] = strt[0, :]
    f_ref[pl.ds(O_EDGE, E)] = edgt[0, :]
    f_ref[pl.ds(O_EDGE + EP, E)] = edgt[1, :]
    rotmt = rotm_ref[...].T
    for c in range(3):
        f_ref[pl.ds(O_CONST + 128 * c, 3)] = rotmt[c, :]
    f_ref[pl.ds(O_CONST + 384, 3)] = foc_ref[...]
    f_ref[pl.ds(O_CONST + 392, 1)] = bias_ref[...]


_pack_call = pl.pallas_call(
    _pack_body,
    out_shape=jax.ShapeDtypeStruct((F_TOTAL,), _F32),
)


def _body(fbuf_h, lene_h, rope_h, refl_h,
          loss_o, c_o, ceq_o, stre_o,
          pos_v, act_v, dir_v, str_v, rope_v, refl_v, edge_v, lene_v,
          consts_v,
          loss_s, c_s, ceq_s, stre_s, sem):
    wid = lax.axis_index("s") * NC + lax.axis_index("c")

    base_r = wid * PER_R
    base_e = wid * PER_E
    base_n = wid * PER_N

    b_refl = jnp.minimum(base_r, R - S_REFL)
    b_lene = jnp.minimum(base_e, E - S_LENE)
    b_edge = jnp.minimum(base_e, EP - S_EDGE)
    b_node = jnp.minimum(base_n, NP - S_NODE)
    b_rope = jnp.minimum(base_n, N - S_ROPE)

    pairs = [
        (fbuf_h.at[pl.ds(O_POS, 3 * NP)], pos_v),
        (fbuf_h.at[pl.ds(O_ACT + b_node, S_NODE)], act_v.at[pl.ds(0, S_NODE)]),
        (fbuf_h.at[pl.ds(O_ACT + NP + b_node, S_NODE)],
         act_v.at[pl.ds(S_NODE, S_NODE)]),
        (fbuf_h.at[pl.ds(O_ACT + 2 * NP + b_node, S_NODE)],
         act_v.at[pl.ds(2 * S_NODE, S_NODE)]),
        (fbuf_h.at[pl.ds(O_DIR + b_node, S_NODE)], dir_v.at[pl.ds(0, S_NODE)]),
        (fbuf_h.at[pl.ds(O_DIR + NP + b_node, S_NODE)],
         dir_v.at[pl.ds(S_NODE, S_NODE)]),
        (fbuf_h.at[pl.ds(O_DIR + 2 * NP + b_node, S_NODE)],
         dir_v.at[pl.ds(2 * S_NODE, S_NODE)]),
        (fbuf_h.at[pl.ds(O_STR + b_node, S_NODE)], str_v),
        (fbuf_h.at[pl.ds(O_EDGE + b_edge, S_EDGE)],
         edge_v.at[pl.ds(0, S_EDGE)]),
        (fbuf_h.at[pl.ds(O_EDGE + EP + b_edge, S_EDGE)],
         edge_v.at[pl.ds(S_EDGE, S_EDGE)]),
        (fbuf_h.at[pl.ds(O_CONST, S_CONST)], consts_v),
        (rope_h.at[pl.ds(b_rope, S_ROPE)], rope_v),
        (lene_h.at[pl.ds(b_lene, S_LENE)], lene_v),
        (refl_h.at[pl.ds(b_refl, S_REFL)], refl_v),
    ]
    handles = [pltpu.async_copy(src, dst, sem) for src, dst in pairs]
    for h in handles:
        h.wait()

    iota = lax.iota(_I32, 16)

    cv0 = consts_v[pl.ds(0, 16)]
    cv1 = consts_v[pl.ds(128, 16)]
    cv2 = consts_v[pl.ds(256, 16)]
    cv3 = consts_v[pl.ds(384, 16)]
    r00, r10, r20 = cv0[0], cv0[1], cv0[2]
    r01, r11, r21 = cv1[0], cv1[1], cv1[2]
    r02, r12, r22 = cv2[0], cv2[1], cv2[2]
    fx, fy, fz = cv3[0], cv3[1], cv3[2]
    bias2 = cv3[8] * 2.0 + 440.0

    def _loss_chunk(j, _):
        ii = jnp.minimum(base_r + j * 16 + iota, R - 1)
        ridx = plsc.load_gather(refl_v, [ii - b_refl])
        px = plsc.load_gather(pos_v, [ridx])
        py = plsc.load_gather(pos_v, [ridx + NP])
        pz = plsc.load_gather(pos_v, [ridx + 2 * NP])
        rx = px * r00 + py * r10 + pz * r20
        ry = px * r01 + py * r11 + pz * r21
        rz = px * r02 + py * r12 + pz * r22
        ex = rx - fx
        ey = ry - fy
        ez = rz - fz
        dis = _sqrt16(ex * ex + ey * ey + ez * ez)
        t = jnp.abs(dis - (rz + bias2)) - 1.0
        loss_s[pl.ds(j * 16, 16)] = jnp.maximum(t, 0.0)
        return 0

    lax.fori_loop(0, PER_R // 16, _loss_chunk, 0, unroll=False)

    def _edge_chunk(j, _):
        ii = jnp.minimum(base_e + j * 16 + iota, E - 1)
        lii = ii - b_edge
        ia = plsc.load_gather(edge_v, [lii]).astype(_I32)
        ib = plsc.load_gather(edge_v, [lii + S_EDGE]).astype(_I32)
        dx = plsc.load_gather(pos_v, [ia]) - plsc.load_gather(pos_v, [ib])
        dy = (plsc.load_gather(pos_v, [ia + NP])
              - plsc.load_gather(pos_v, [ib + NP]))
        dz = (plsc.load_gather(pos_v, [ia + 2 * NP])
              - plsc.load_gather(pos_v, [ib + 2 * NP]))
        lens = _sqrt16(dx * dx + dy * dy + dz * dz)
        le = plsc.load_gather(lene_v, [ii - b_lene])
        c = jnp.maximum(jnp.abs(lens - le) - 0.007 * le, 0.0) * 100.0
        c_s[pl.ds(j * 16, 16)] = c
        return 0

    lax.fori_loop(0, PER_E // 16, _edge_chunk, 0, unroll=False)

    def _node_chunk(j, _):
        ii = jnp.minimum(base_n + j * 16 + iota, N - 1)
        lii = ii - b_node
        s = plsc.load_gather(str_v, [lii])
        rx = (plsc.load_gather(act_v, [lii])
              + plsc.load_gather(dir_v, [lii]) * s
              - plsc.load_gather(pos_v, [ii]))
        ry = (plsc.load_gather(act_v, [lii + S_NODE])
              + plsc.load_gather(dir_v, [lii + S_NODE]) * s
              - plsc.load_gather(pos_v, [ii + NP]))
        rz = (plsc.load_gather(act_v, [lii + 2 * S_NODE])
              + plsc.load_gather(dir_v, [lii + 2 * S_NODE]) * s
              - plsc.load_gather(pos_v, [ii + 2 * NP]))
        nn = _sqrt16(rx * rx + ry * ry + rz * rz)
        lr = plsc.load_gather(rope_v, [ii - b_rope])
        ceq_s[pl.ds(j * 16, 16)] = jnp.abs(lr - nn) * 100.0
        stre_s[pl.ds(j * 16, 16)] = jnp.maximum(jnp.abs(s) - 0.6, 0.0)
        return 0

    lax.fori_loop(0, PER_N // 16, _node_chunk, 0, unroll=False)

    pltpu.sync_copy(loss_s, loss_o.at[pl.ds(base_r, PER_R)])
    pltpu.sync_copy(c_s, c_o.at[pl.ds(base_e, PER_E)])
    pltpu.sync_copy(ceq_s, ceq_o.at[pl.ds(base_n, PER_N)])
    pltpu.sync_copy(stre_s, stre_o.at[pl.ds(base_n, PER_N)])


_sc_call = functools.partial(
    pl.kernel,
    out_type=[
        jax.ShapeDtypeStruct((NW * PER_R,), _F32),
        jax.ShapeDtypeStruct((NW * PER_E,), _F32),
        jax.ShapeDtypeStruct((NW * PER_N,), _F32),
        jax.ShapeDtypeStruct((NW * PER_N,), _F32),
    ],
    mesh=plsc.VectorSubcoreMesh(core_axis_name="c", subcore_axis_name="s",
                                num_cores=NC, num_subcores=NS),
    compiler_params=pltpu.CompilerParams(needs_layout_passes=False),
    scratch_types=[
        pltpu.VMEM((3 * NP,), _F32),
        pltpu.VMEM((3 * S_NODE,), _F32),
        pltpu.VMEM((3 * S_NODE,), _F32),
        pltpu.VMEM((S_NODE,), _F32),
        pltpu.VMEM((S_ROPE,), _F32),
        pltpu.VMEM((S_REFL,), _I32),
        pltpu.VMEM((2 * S_EDGE,), _F32),
        pltpu.VMEM((S_LENE,), _F32),
        pltpu.VMEM((S_CONST,), _F32),
        pltpu.VMEM((PER_R,), _F32),
        pltpu.VMEM((PER_E,), _F32),
        pltpu.VMEM((PER_N,), _F32),
        pltpu.VMEM((PER_N,), _F32),
        pltpu.SemaphoreType.DMA,
    ],
)(_body)


def kernel(pos, stretch, bias, rotm, direction, focus, len_edges, act_up,
           len_rope, refl_idx, all_edges):
    fbuf = _pack_call(pos, stretch, bias, rotm, focus,
                      act_up, direction, all_edges.astype(_I32))
    loss_p, c_p, ceq_p, stre_p = _sc_call(
        fbuf, len_edges, len_rope, refl_idx.astype(_I32))
    return jnp.concatenate([loss_p[:R], c_p[:E], ceq_p[:N], stre_p[:N]])
